# baseline (speedup 1.0000x reference)
import jax
import jax.numpy as jnp
from jax import lax
from jax.experimental import pallas as pl
from jax.experimental.pallas import tpu as pltpu
from jax.experimental.pallas import tpu_sc as plsc

_NUM_CORES = 2
_NUM_SUBCORES = 16
_NUM_WORKERS = _NUM_CORES * _NUM_SUBCORES
_CHUNK = 200
_DEPTH = 4


def kernel(x, weight):
    b, h = x.shape
    n = b * h
    dim = weight.shape[1]
    idx_per_w = n // _NUM_WORKERS
    n_chunks = idx_per_w // _CHUNK
    idx = x.T.reshape(n)

    mesh = plsc.VectorSubcoreMesh(core_axis_name="c", subcore_axis_name="s")

    @pl.kernel(
        out_type=jax.ShapeDtypeStruct((n, dim), weight.dtype),
        mesh=mesh,
        scratch_types=[pltpu.VMEM((idx_per_w,), jnp.int32)]
        + [pltpu.VMEM((_CHUNK, dim), jnp.float32)] * _DEPTH
        + [pltpu.SemaphoreType.DMA] * (2 * _DEPTH),
    )
    def gather_kernel(w_hbm, i_hbm, o_hbm, idx_v, *bufs_sems):
        bufs = bufs_sems[:_DEPTH]
        semg = bufs_sems[_DEPTH : 2 * _DEPTH]
        semw = bufs_sems[2 * _DEPTH :]
        wid = lax.axis_index("s") * _NUM_CORES + lax.axis_index("c")
        base = wid * idx_per_w
        pltpu.sync_copy(i_hbm.at[pl.ds(base, idx_per_w)], idx_v)

        def gather_start(c, k):
            pltpu.async_copy(
                w_hbm.at[idx_v.at[pl.ds(c * _CHUNK, _CHUNK)]], bufs[k], semg[k]
            )

        def gather_wait(c, k):
            pltpu.make_async_copy(
                w_hbm.at[idx_v.at[pl.ds(c * _CHUNK, _CHUNK)]], bufs[k], semg[k]
            ).wait()

        def wb_start(c, k):
            pltpu.async_copy(
                bufs[k], o_hbm.at[pl.ds(base + c * _CHUNK, _CHUNK)], semw[k]
            )

        def wb_drain(c, k):
            pltpu.make_async_copy(
                bufs[k], o_hbm.at[pl.ds(base + c * _CHUNK, _CHUNK)], semw[k]
            ).wait()

        for k in range(_DEPTH):
            gather_start(k, k)

        @pl.loop(0, n_chunks, step=_DEPTH)
        def _(c):
            for k in range(_DEPTH):
                gather_wait(c + k, k)
                wb_start(c + k, k)
            for k in range(_DEPTH):

                @pl.when(c + k + _DEPTH < n_chunks)
                def _(k=k):
                    wb_drain(c + k, k)
                    gather_start(c + k + _DEPTH, k)

        for k in range(_DEPTH):
            wb_drain(n_chunks - _DEPTH + k, k)

    out = gather_kernel(weight, idx)
    return out.reshape(h, b, dim).transpose(1, 0, 2)


# speedup vs baseline: 1.0128x; 1.0128x over previous
import jax
import jax.numpy as jnp
from jax import lax
from jax.experimental import pallas as pl
from jax.experimental.pallas import tpu as pltpu
from jax.experimental.pallas import tpu_sc as plsc

_NUM_CORES = 2
_NUM_SUBCORES = 16
_NUM_WORKERS = _NUM_CORES * _NUM_SUBCORES
_CHUNK = 400


def kernel(x, weight):
    b, h = x.shape
    n = b * h
    dim = weight.shape[1]
    idx_per_w = n // _NUM_WORKERS
    n_chunks = idx_per_w // _CHUNK
    idx = x.T.reshape(n)

    mesh = plsc.VectorSubcoreMesh(core_axis_name="c", subcore_axis_name="s")

    @pl.kernel(
        out_type=jax.ShapeDtypeStruct((n, dim), weight.dtype),
        mesh=mesh,
        scratch_types=[
            pltpu.VMEM((idx_per_w,), jnp.int32),
            pltpu.VMEM((_CHUNK, dim), jnp.float32),
            pltpu.VMEM((_CHUNK, dim), jnp.float32),
            pltpu.SemaphoreType.DMA,
            pltpu.SemaphoreType.DMA,
            pltpu.SemaphoreType.DMA,
            pltpu.SemaphoreType.DMA,
        ],
    )
    def gather_kernel(
        w_hbm, i_hbm, o_hbm, idx_v, rows_v0, rows_v1, sem0, sem1, semw0, semw1
    ):
        wid = lax.axis_index("s") * _NUM_CORES + lax.axis_index("c")
        base = wid * idx_per_w
        pltpu.sync_copy(i_hbm.at[pl.ds(base, idx_per_w)], idx_v)

        def gather_start(c, buf, sem):
            pltpu.async_copy(
                w_hbm.at[idx_v.at[pl.ds(c * _CHUNK, _CHUNK)]], buf, sem
            )

        def gather_wait(c, buf, sem):
            pltpu.make_async_copy(
                w_hbm.at[idx_v.at[pl.ds(c * _CHUNK, _CHUNK)]], buf, sem
            ).wait()

        def wb_start(c, buf, sem):
            pltpu.async_copy(buf, o_hbm.at[pl.ds(base + c * _CHUNK, _CHUNK)], sem)

        def wb_drain(c, buf, sem):
            pltpu.make_async_copy(
                buf, o_hbm.at[pl.ds(base + c * _CHUNK, _CHUNK)], sem
            ).wait()

        gather_start(0, rows_v0, sem0)
        gather_start(1, rows_v1, sem1)

        @pl.loop(0, n_chunks, step=2)
        def _(c):
            gather_wait(c, rows_v0, sem0)
            wb_start(c, rows_v0, semw0)
            gather_wait(c + 1, rows_v1, sem1)
            wb_start(c + 1, rows_v1, semw1)
            wb_drain(c, rows_v0, semw0)

            @pl.when(c + 2 < n_chunks)
            def _():
                gather_start(c + 2, rows_v0, sem0)

            wb_drain(c + 1, rows_v1, semw1)

            @pl.when(c + 3 < n_chunks)
            def _():
                gather_start(c + 3, rows_v1, sem1)

    out = gather_kernel(weight, idx)
    return out.reshape(h, b, dim).transpose(1, 0, 2)


# speedup vs baseline: 1.0567x; 1.0434x over previous
import jax
import jax.numpy as jnp
from jax import lax
from jax.experimental import pallas as pl
from jax.experimental.pallas import tpu as pltpu
from jax.experimental.pallas import tpu_sc as plsc

_NUM_CORES = 2
_NUM_SUBCORES = 16
_NUM_WORKERS = _NUM_CORES * _NUM_SUBCORES
_CHUNK = 320


def kernel(x, weight):
    b, h = x.shape
    n = b * h
    dim = weight.shape[1]
    idx_per_w = n // _NUM_WORKERS
    n_chunks = idx_per_w // _CHUNK
    idx = x.T.reshape(n)

    mesh = plsc.VectorSubcoreMesh(core_axis_name="c", subcore_axis_name="s")

    @pl.kernel(
        out_type=jax.ShapeDtypeStruct((n, dim), weight.dtype),
        mesh=mesh,
        scratch_types=[
            pltpu.VMEM((idx_per_w,), jnp.int32),
            pltpu.VMEM((_CHUNK, dim), jnp.float32),
            pltpu.VMEM((_CHUNK, dim), jnp.float32),
            pltpu.SemaphoreType.DMA,
            pltpu.SemaphoreType.DMA,
            pltpu.SemaphoreType.DMA,
            pltpu.SemaphoreType.DMA,
        ],
    )
    def gather_kernel(
        w_hbm, i_hbm, o_hbm, idx_v, rows_v0, rows_v1, sem0, sem1, semw0, semw1
    ):
        wid = lax.axis_index("s") * _NUM_CORES + lax.axis_index("c")
        base = wid * idx_per_w
        pltpu.sync_copy(i_hbm.at[pl.ds(base, idx_per_w)], idx_v)

        def gather_start(c, buf, sem):
            pltpu.async_copy(
                w_hbm.at[idx_v.at[pl.ds(c * _CHUNK, _CHUNK)]], buf, sem
            )

        def gather_wait(c, buf, sem):
            pltpu.make_async_copy(
                w_hbm.at[idx_v.at[pl.ds(c * _CHUNK, _CHUNK)]], buf, sem
            ).wait()

        def wb_start(c, buf, sem):
            pltpu.async_copy(buf, o_hbm.at[pl.ds(base + c * _CHUNK, _CHUNK)], sem)

        def wb_drain(c, buf, sem):
            pltpu.make_async_copy(
                buf, o_hbm.at[pl.ds(base + c * _CHUNK, _CHUNK)], sem
            ).wait()

        gather_start(0, rows_v0, sem0)
        gather_start(1, rows_v1, sem1)

        @pl.loop(0, n_chunks, step=2)
        def _(c):
            gather_wait(c, rows_v0, sem0)
            wb_start(c, rows_v0, semw0)
            wb_drain(c, rows_v0, semw0)

            @pl.when(c + 2 < n_chunks)
            def _():
                gather_start(c + 2, rows_v0, sem0)

            gather_wait(c + 1, rows_v1, sem1)
            wb_start(c + 1, rows_v1, semw1)
            wb_drain(c + 1, rows_v1, semw1)

            @pl.when(c + 3 < n_chunks)
            def _():
                gather_start(c + 3, rows_v1, sem1)

    out = gather_kernel(weight, idx)
    return out.reshape(h, b, dim).transpose(1, 0, 2)
